# Initial kernel scaffold; baseline (speedup 1.0000x reference)
import jax
import jax.numpy as jnp
from jax.experimental import pallas as pl


def kernel(xyz, features, rand_idx, w1, g1, be1, w2, g2, be2, mw1, mb1, mg1, mbe1, mw2, mb2, mg2, mbe2, mw3, mb3):
    raise NotImplementedError("write your pallas kernel here")


# speedup vs baseline: 9.1438x; 9.1438x over previous
import functools

import jax
import jax.numpy as jnp
from jax import lax
from jax.experimental import pallas as pl
from jax.experimental.pallas import tpu as pltpu
from jax.experimental.pallas import tpu_sc as plsc

B, N, M, K = 4, 8192, 1024, 16
C_IN = 32
C0 = C_IN + 5
C1, C2 = 64, 128
D = 48
MT = 128
MQ = 128
PT = MQ * K
NQ = M // MQ
CNT1 = float(B * M * K)
CNT2 = float(B * M)
_NW = 32


def _make_gather(nrows):
    bpw = nrows // _NW
    mesh = plsc.VectorSubcoreMesh(core_axis_name="c", subcore_axis_name="s")

    @functools.partial(
        pl.kernel, mesh=mesh,
        out_type=jax.ShapeDtypeStruct((nrows, D), jnp.float32),
        compiler_params=pltpu.CompilerParams(use_tc_tiling_on_sc=False),
        scratch_types=[
            pltpu.VMEM((bpw,), jnp.int32),
            pltpu.VMEM((bpw, D), jnp.float32),
            pltpu.SemaphoreType.DMA,
        ],
    )
    def gk(table_hbm, idx_hbm, out_hbm, idx_v, rows_v, sem):
        wid = lax.axis_index("s") * 2 + lax.axis_index("c")
        base = wid * bpw
        pltpu.sync_copy(idx_hbm.at[pl.ds(base, bpw)], idx_v)
        pltpu.async_copy(table_hbm.at[idx_v], rows_v, sem).wait()
        pltpu.sync_copy(rows_v, out_hbm.at[pl.ds(base, bpw)])

    return gk


def _gather_rows(table, idx, nrows):
    return _make_gather(nrows)(table, idx)


def _knn_body(xyz_ref, q_ref, idx_ref):
    P = xyz_ref[0]
    qrows = q_ref[0]
    q = qrows[:, 0:3]
    q2 = jnp.sum(q * q, axis=1, keepdims=True)
    p2 = jnp.sum(P * P, axis=0, keepdims=True)
    qp = jnp.dot(q, P, preferred_element_type=jnp.float32)
    d2 = q2 + p2 - 2.0 * qp
    iota = lax.broadcasted_iota(jnp.int32, (MT, N), 1)
    kiota = lax.broadcasted_iota(jnp.int32, (MT, K), 1)
    big_f = jnp.float32(3.0e38)
    big_i = jnp.int32(2**30)

    def step(k, carry):
        d2, prev, cols = carry
        d2 = jnp.where(iota == prev, big_f, d2)
        v = jnp.min(d2, axis=1, keepdims=True)
        cand = jnp.where(d2 == v, iota, big_i)
        icol = jnp.min(cand, axis=1, keepdims=True)
        cols = jnp.where(kiota == k, icol, cols)
        return d2, icol, cols

    _, _, cols = lax.fori_loop(
        0, K, step,
        (d2, jnp.full((MT, 1), -1, jnp.int32),
         jnp.zeros((MT, K), jnp.int32)))
    idx_ref[0] = cols


def _knn(xyz_cm, sampled):
    return pl.pallas_call(
        _knn_body,
        grid=(B, M // MT),
        in_specs=[
            pl.BlockSpec((1, 3, N), lambda b, q: (b, 0, 0)),
            pl.BlockSpec((1, MT, D), lambda b, q: (b, q, 0)),
        ],
        out_specs=pl.BlockSpec((1, MT, K), lambda b, q: (b, q, 0)),
        out_shape=jax.ShapeDtypeStruct((B, M, K), jnp.int32),
    )(xyz_cm, sampled)


def _pass_c_body(r_ref, s_ref, g_ref, st_ref, acc, sums):
    lin = pl.program_id(0) * NQ + pl.program_id(1)

    @pl.when(lin == 0)
    def _():
        acc[...] = jnp.zeros_like(acc)
        sums[...] = jnp.zeros_like(sums)

    r3 = r_ref[0].reshape(MQ, K, D)
    srows = s_ref[0]
    rela = r3[:, :, 0:3] - srows[:, None, 0:3]
    ds = jnp.sqrt(jnp.sum(rela * rela, axis=-1, keepdims=True) + 1e-12)
    dens = 1.0 / (jnp.mean(ds, axis=1, keepdims=True) + 1e-6)
    densb = jnp.broadcast_to(dens, (MQ, K, 1))
    g3 = jnp.concatenate([rela, ds, densb, r3[:, :, 3:3 + C_IN]], axis=-1)
    g2d = g3.reshape(PT, C0)
    gp = jnp.concatenate([g2d, jnp.zeros((PT, D - C0), jnp.float32)], axis=1)
    acc[...] += lax.dot_general(gp, gp, (((0,), (0,)), ((), ())),
                                preferred_element_type=jnp.float32)
    sums[...] += jnp.sum(gp, axis=0, keepdims=True)
    gt = jnp.transpose(gp)
    g_ref[0] = gt[0:C0, :]

    @pl.when(lin == B * NQ - 1)
    def _():
        st_ref[...] = jnp.concatenate([acc[...], sums[...]], axis=0)


def _pass_c(rows1, sampled):
    return pl.pallas_call(
        _pass_c_body,
        grid=(B, NQ),
        in_specs=[
            pl.BlockSpec((1, MQ, K * D), lambda b, q: (b, q, 0)),
            pl.BlockSpec((1, MQ, D), lambda b, q: (b, q, 0)),
        ],
        out_specs=[
            pl.BlockSpec((1, C0, PT), lambda b, q: (b, 0, q)),
            pl.BlockSpec((D + 1, D), lambda b, q: (0, 0)),
        ],
        out_shape=[
            jax.ShapeDtypeStruct((B, C0, M * K), jnp.float32),
            jax.ShapeDtypeStruct((D + 1, D), jnp.float32),
        ],
        scratch_shapes=[
            pltpu.VMEM((D, D), jnp.float32),
            pltpu.VMEM((1, D), jnp.float32),
        ],
    )(rows1, sampled)


def _bn1_coeffs(st1, w1p, g1c, be1c):
    mean_g = st1[D:D + 1, :] / CNT1
    m2 = st1[0:D, :] / CNT1
    cov = m2 - lax.dot_general(mean_g, mean_g, (((0,), (0,)), ((), ())),
                               preferred_element_type=jnp.float32)
    t = jnp.dot(w1p, cov, preferred_element_type=jnp.float32)
    var = jnp.sum(t * w1p, axis=1, keepdims=True)
    mean = lax.dot_general(w1p, mean_g, (((1,), (1,)), ((), ())),
                           preferred_element_type=jnp.float32)
    scale = g1c * lax.rsqrt(var + 1e-5)
    shift = be1c - mean * scale
    return scale, shift


def _bn_next_coeffs(stats, w, gc, bc, cnt, c_in):
    mean_a = stats[c_in:c_in + 1, :] / cnt
    m2 = stats[0:c_in, :] / cnt
    cov = m2 - lax.dot_general(mean_a, mean_a, (((0,), (0,)), ((), ())),
                               preferred_element_type=jnp.float32)
    t = jnp.dot(w, cov, preferred_element_type=jnp.float32)
    var = jnp.sum(t * w, axis=1, keepdims=True)
    mean = lax.dot_general(w, mean_a, (((1,), (1,)), ((), ())),
                           preferred_element_type=jnp.float32)
    scale = gc * lax.rsqrt(var + 1e-5)
    shift = bc - mean * scale
    return scale, shift


def _pass_d_body(g_ref, st1_ref, w1p_ref, g1_ref, be1_ref, st2_ref, acc, sums):
    lin = pl.program_id(0) * NQ + pl.program_id(1)

    @pl.when(lin == 0)
    def _():
        acc[...] = jnp.zeros_like(acc)
        sums[...] = jnp.zeros_like(sums)

    w1p = w1p_ref[...]
    scale1, shift1 = _bn1_coeffs(st1_ref[...], w1p, g1_ref[...], be1_ref[...])
    g = g_ref[0]
    h = jnp.dot(w1p[:, 0:C0], g, preferred_element_type=jnp.float32)
    a1 = jnp.maximum(h * scale1 + shift1, 0.0)
    acc[...] += lax.dot_general(a1, a1, (((1,), (1,)), ((), ())),
                                preferred_element_type=jnp.float32)
    sums[...] += jnp.sum(a1, axis=1, keepdims=True)

    @pl.when(lin == B * NQ - 1)
    def _():
        st2_ref[...] = jnp.concatenate(
            [acc[...], jnp.transpose(sums[...])], axis=0)


def _pass_d(grouped3, st1, w1p, g1c, be1c):
    return pl.pallas_call(
        _pass_d_body,
        grid=(B, NQ),
        in_specs=[
            pl.BlockSpec((1, C0, PT), lambda b, q: (b, 0, q)),
            pl.BlockSpec((D + 1, D), lambda b, q: (0, 0)),
            pl.BlockSpec((C1, D), lambda b, q: (0, 0)),
            pl.BlockSpec((C1, 1), lambda b, q: (0, 0)),
            pl.BlockSpec((C1, 1), lambda b, q: (0, 0)),
        ],
        out_specs=pl.BlockSpec((C1 + 1, C1), lambda b, q: (0, 0)),
        out_shape=jax.ShapeDtypeStruct((C1 + 1, C1), jnp.float32),
        scratch_shapes=[
            pltpu.VMEM((C1, C1), jnp.float32),
            pltpu.VMEM((C1, 1), jnp.float32),
        ],
    )(grouped3, st1, w1p, g1c, be1c)


def _pass_e_body(g_ref, s_ref, sel_ref, st1_ref, st2_ref, w1p_ref, g1_ref,
                 be1_ref, w2_ref, g2_ref, be2_ref,
                 kp_ref, afm_ref, af_ref, ald_ref, st3_ref, acc, sums):
    lin = pl.program_id(0) * NQ + pl.program_id(1)

    @pl.when(lin == 0)
    def _():
        acc[...] = jnp.zeros_like(acc)
        sums[...] = jnp.zeros_like(sums)

    w1p = w1p_ref[...]
    w2 = w2_ref[...]
    scale1, shift1 = _bn1_coeffs(st1_ref[...], w1p, g1_ref[...], be1_ref[...])
    scale2, shift2 = _bn_next_coeffs(st2_ref[...], w2, g2_ref[...],
                                     be2_ref[...], CNT1, C1)
    g = g_ref[0]
    h = jnp.dot(w1p[:, 0:C0], g, preferred_element_type=jnp.float32)
    a1 = jnp.maximum(h * scale1 + shift1, 0.0)
    h2 = jnp.dot(w2, a1, preferred_element_type=jnp.float32)
    emb = jnp.maximum(h2 * scale2 + shift2, 0.0)
    sel = sel_ref[...]
    x1 = jnp.max(emb, axis=0, keepdims=True)
    e = jnp.exp(x1)
    ssum = jnp.dot(e, sel, preferred_element_type=jnp.float32)
    sb = lax.dot_general(ssum, sel, (((1,), (1,)), ((), ())),
                         preferred_element_type=jnp.float32)
    aw_row = e / sb
    afm = emb * aw_row
    afm_ref[0] = afm
    af2 = jnp.dot(afm, sel, preferred_element_type=jnp.float32)
    af_ref[0] = af2
    acc[...] += lax.dot_general(af2, af2, (((1,), (1,)), ((), ())),
                                preferred_element_type=jnp.float32)
    sums[...] += jnp.sum(af2, axis=1, keepdims=True)

    srows = s_ref[0]
    kxyzd = jnp.dot(g[0:8, :] * aw_row, sel,
                    preferred_element_type=jnp.float32)
    kt = jnp.transpose(kxyzd)
    kp_ref[0] = kt[:, 0:3] + srows[:, 0:3]
    ald_ref[0] = kt[:, 4:5]

    @pl.when(lin == B * NQ - 1)
    def _():
        st3_ref[...] = jnp.concatenate(
            [acc[...], jnp.transpose(sums[...])], axis=0)


def _pass_e(grouped3, sampled, sel, st1, st2, w1p, g1c, be1c, w2, g2c, be2c):
    return pl.pallas_call(
        _pass_e_body,
        grid=(B, NQ),
        in_specs=[
            pl.BlockSpec((1, C0, PT), lambda b, q: (b, 0, q)),
            pl.BlockSpec((1, MQ, D), lambda b, q: (b, q, 0)),
            pl.BlockSpec((PT, MQ), lambda b, q: (0, 0)),
            pl.BlockSpec((D + 1, D), lambda b, q: (0, 0)),
            pl.BlockSpec((C1 + 1, C1), lambda b, q: (0, 0)),
            pl.BlockSpec((C1, D), lambda b, q: (0, 0)),
            pl.BlockSpec((C1, 1), lambda b, q: (0, 0)),
            pl.BlockSpec((C1, 1), lambda b, q: (0, 0)),
            pl.BlockSpec((C2, C1), lambda b, q: (0, 0)),
            pl.BlockSpec((C2, 1), lambda b, q: (0, 0)),
            pl.BlockSpec((C2, 1), lambda b, q: (0, 0)),
        ],
        out_specs=[
            pl.BlockSpec((1, MQ, 3), lambda b, q: (b, q, 0)),
            pl.BlockSpec((1, C2, PT), lambda b, q: (b, 0, q)),
            pl.BlockSpec((1, C2, MQ), lambda b, q: (b, 0, q)),
            pl.BlockSpec((1, MQ, 1), lambda b, q: (b, q, 0)),
            pl.BlockSpec((C2 + 1, C2), lambda b, q: (0, 0)),
        ],
        out_shape=[
            jax.ShapeDtypeStruct((B, M, 3), jnp.float32),
            jax.ShapeDtypeStruct((B, C2, M * K), jnp.float32),
            jax.ShapeDtypeStruct((B, C2, M), jnp.float32),
            jax.ShapeDtypeStruct((B, M, 1), jnp.float32),
            jax.ShapeDtypeStruct((C2 + 1, C2), jnp.float32),
        ],
        scratch_shapes=[
            pltpu.VMEM((C2, C2), jnp.float32),
            pltpu.VMEM((C2, 1), jnp.float32),
        ],
    )(grouped3, sampled, sel, st1, st2, w1p, g1c, be1c, w2, g2c, be2c)


def _pass_f_body(af_ref, st3_ref, mw1_ref, mb1_ref, mg1_ref, mbe1_ref,
                 mw2_ref, mb2_ref, mg2_ref, mbe2_ref, mw3_ref, mb3_ref,
                 sig_ref, acc, sums):
    p = pl.program_id(0)
    b = pl.program_id(1)

    @pl.when((p == 0) & (b == 0))
    def _():
        acc[...] = jnp.zeros_like(acc)
        sums[...] = jnp.zeros_like(sums)

    mw1 = mw1_ref[...]
    scale3, shift3 = _bn_next_coeffs(st3_ref[...], mw1, mg1_ref[...],
                                     mbe1_ref[...], CNT2, C2)
    af = af_ref[0]
    x = jnp.dot(mw1, af, preferred_element_type=jnp.float32) + mb1_ref[...]
    mean_corr = mb1_ref[...] * scale3
    s1 = jnp.maximum(x * scale3 + shift3 - mean_corr, 0.0)

    @pl.when(p == 0)
    def _():
        acc[...] += lax.dot_general(s1, s1, (((1,), (1,)), ((), ())),
                                    preferred_element_type=jnp.float32)
        sums[...] += jnp.sum(s1, axis=1, keepdims=True)

    @pl.when(p == 1)
    def _():
        mean_s = sums[...] / CNT2
        m2 = acc[...] / CNT2
        cov = m2 - lax.dot_general(mean_s, mean_s, (((1,), (1,)), ((), ())),
                                   preferred_element_type=jnp.float32)
        mw2 = mw2_ref[...]
        t = jnp.dot(mw2, cov, preferred_element_type=jnp.float32)
        var = jnp.sum(t * mw2, axis=1, keepdims=True)
        mean = jnp.dot(mw2, mean_s, preferred_element_type=jnp.float32) \
            + mb2_ref[...]
        scale4 = mg2_ref[...] * lax.rsqrt(var + 1e-5)
        shift4 = mbe2_ref[...] - mean * scale4
        x2 = jnp.dot(mw2, s1, preferred_element_type=jnp.float32) \
            + mb2_ref[...]
        s2 = jnp.maximum(x2 * scale4 + shift4, 0.0)
        s3 = jnp.dot(mw3_ref[...], s2, preferred_element_type=jnp.float32)
        s3 = s3[0:1, :] + mb3_ref[...]
        sp = jnp.where(s3 > 20.0,
                       s3,
                       jnp.log(1.0 + jnp.exp(jnp.minimum(s3, 20.0))))
        sig_ref[0] = sp + 0.001


def _pass_f(af, st3, mw1, mb1c, mg1c, mbe1c, mw2, mb2c, mg2c, mbe2c,
            mw3p, mb3c):
    return pl.pallas_call(
        _pass_f_body,
        grid=(2, B),
        in_specs=[
            pl.BlockSpec((1, C2, M), lambda p, b: (b, 0, 0)),
            pl.BlockSpec((C2 + 1, C2), lambda p, b: (0, 0)),
            pl.BlockSpec((C2, C2), lambda p, b: (0, 0)),
            pl.BlockSpec((C2, 1), lambda p, b: (0, 0)),
            pl.BlockSpec((C2, 1), lambda p, b: (0, 0)),
            pl.BlockSpec((C2, 1), lambda p, b: (0, 0)),
            pl.BlockSpec((C2, C2), lambda p, b: (0, 0)),
            pl.BlockSpec((C2, 1), lambda p, b: (0, 0)),
            pl.BlockSpec((C2, 1), lambda p, b: (0, 0)),
            pl.BlockSpec((C2, 1), lambda p, b: (0, 0)),
            pl.BlockSpec((8, C2), lambda p, b: (0, 0)),
            pl.BlockSpec((1, 1), lambda p, b: (0, 0)),
        ],
        out_specs=pl.BlockSpec((1, 1, M), lambda p, b: (b, 0, 0)),
        out_shape=jax.ShapeDtypeStruct((B, 1, M), jnp.float32),
        scratch_shapes=[
            pltpu.VMEM((C2, C2), jnp.float32),
            pltpu.VMEM((C2, 1), jnp.float32),
        ],
    )(af, st3, mw1, mb1c, mg1c, mbe1c, mw2, mb2c, mg2c, mbe2c, mw3p, mb3c)


def kernel(xyz, features, rand_idx, w1, g1, be1, w2, g2, be2,
           mw1, mb1, mg1, mbe1, mw2, mb2, mg2, mbe2, mw3, mb3):
    f32 = jnp.float32
    feats_t = jnp.transpose(features, (0, 2, 1))
    table = jnp.concatenate(
        [xyz, feats_t, jnp.zeros((B, N, D - 3 - C_IN), f32)],
        axis=-1).reshape(B * N, D)
    offs = (jnp.arange(B, dtype=jnp.int32) * N)[:, None]
    idx0 = (rand_idx.astype(jnp.int32)[None, :] + offs).reshape(-1)
    sampled = _gather_rows(table, idx0, B * M).reshape(B, M, D)

    xyz_cm = jnp.transpose(xyz, (0, 2, 1))
    idx = _knn(xyz_cm, sampled)

    idx1 = (idx.reshape(B, M * K) + offs).reshape(-1)
    rows1 = _gather_rows(table, idx1, B * M * K)
    rows1 = rows1.reshape(B, M, K * D)

    grouped3, st1 = _pass_c(rows1, sampled)

    w1p = jnp.concatenate([w1, jnp.zeros((C1, D - C0), f32)], axis=1)
    g1c = g1.reshape(C1, 1)
    be1c = be1.reshape(C1, 1)
    st2 = _pass_d(grouped3, st1, w1p, g1c, be1c)

    sel = (jnp.arange(PT, dtype=jnp.int32)[:, None] // K
           == jnp.arange(MQ, dtype=jnp.int32)[None, :]).astype(f32)
    kp, afm, af, ald, st3 = _pass_e(
        grouped3, sampled, sel, st1, st2, w1p, g1c, be1c,
        w2, g2.reshape(C2, 1), be2.reshape(C2, 1))

    mw3p = jnp.concatenate([mw3, jnp.zeros((7, C2), f32)], axis=0)
    sig = _pass_f(af, st3, mw1, mb1.reshape(C2, 1), mg1.reshape(C2, 1),
                  mbe1.reshape(C2, 1), mw2, mb2.reshape(C2, 1),
                  mg2.reshape(C2, 1), mbe2.reshape(C2, 1),
                  mw3p, mb3.reshape(1, 1))

    grouped_features = grouped3.reshape(B, C0, M, K)
    attentive_feature_map = afm.reshape(B, C2, M, K)
    return (kp, sig[:, 0, :], af, grouped_features,
            attentive_feature_map, ald[:, :, 0])
